# baseline (speedup 1.0000x reference)
import functools

import jax
import jax.numpy as jnp
from jax import lax
from jax.experimental import pallas as pl
from jax.experimental.pallas import tpu as pltpu
from jax.experimental.pallas import tpu_sc as plsc

BLK = 256
FBLK = 1024


def _router_body(x_ref, gw_ref, pos_ref, pval_ref, gids_ref, nblk_ref,
                 aux_ref, xb16_ref, *, num_experts, nb_max):
    x = x_ref[...]
    xb16_ref[...] = x.astype(jnp.bfloat16)
    s = x.shape[0]
    logits = lax.dot_general(x, gw_ref[...], (((1,), (1,)), ((), ())),
                             preferred_element_type=jnp.float32)
    iota_e = lax.broadcasted_iota(jnp.int32, (s, num_experts), 1)
    big = jnp.int32(num_experts + 1)
    m1 = jnp.max(logits, axis=1, keepdims=True)
    i1 = jnp.min(jnp.where(logits == m1, iota_e, big), axis=1, keepdims=True)
    masked = jnp.where(iota_e == i1, -jnp.inf, logits)
    m2 = jnp.max(masked, axis=1, keepdims=True)
    i2 = jnp.min(jnp.where(masked == m2, iota_e, big), axis=1, keepdims=True)
    p1 = jax.nn.sigmoid(m1 - m2)
    p2 = 1.0 - p1

    probs = jax.nn.softmax(logits, axis=1)
    colsum = jnp.sum(probs, axis=0)
    aux_ref[0, 0] = num_experts * jnp.sum(colsum * colsum) / (s * s)

    ind = (jnp.where(iota_e == i1, 1.0, 0.0)
           + jnp.where(iota_e == i2, 1.0, 0.0))
    ia = lax.broadcasted_iota(jnp.int32, (s, s), 0)
    ib = lax.broadcasted_iota(jnp.int32, (s, s), 1)
    ltri = jnp.where(ib < ia, 1.0, 0.0)
    rank = lax.dot_general(ltri, ind, (((1,), (0,)), ((), ())),
                           preferred_element_type=jnp.float32)
    cnt = jnp.sum(ind, axis=0, keepdims=True).astype(jnp.int32)
    blocks = (cnt + (BLK - 1)) // BLK
    ee = lax.broadcasted_iota(jnp.int32, (num_experts, num_experts), 0)
    ep = lax.broadcasted_iota(jnp.int32, (num_experts, num_experts), 1)
    excl = jnp.sum(jnp.where(ep < ee, blocks, 0), axis=1,
                   keepdims=True)
    off_rows = (excl.reshape(1, num_experts) * BLK).astype(jnp.float32)
    slot = off_rows + rank
    pos1 = jnp.sum(jnp.where(iota_e == i1, slot, 0.0), axis=1,
                   keepdims=True)
    pos2 = jnp.sum(jnp.where(iota_e == i2, slot, 0.0), axis=1,
                   keepdims=True)
    pos_ref[...] = jnp.concatenate(
        [pos1.reshape(1, s), pos2.reshape(1, s)], axis=0).astype(jnp.int32)
    pval_ref[...] = jnp.concatenate(
        [p1.reshape(1, s), p2.reshape(1, s)], axis=0)

    nblk_ref[0, 0] = jnp.sum(blocks)
    ends = (excl.reshape(num_experts, 1)
            + blocks.reshape(num_experts, 1))
    bb = lax.broadcasted_iota(jnp.int32, (num_experts, nb_max), 1)
    g = jnp.sum(jnp.where(bb >= ends, 1, 0), axis=0, keepdims=True)
    gids_ref[...] = jnp.minimum(g, num_experts - 1)


def _scatter_body(x_hbm, pos_hbm, xs_hbm, rows_v, idx0_v, idx1_v,
                  sem0, sem1, *, tok_per_w, num_cores):
    wid = lax.axis_index("s") * num_cores + lax.axis_index("c")
    base = wid * tok_per_w
    pltpu.sync_copy(pos_hbm.at[0, pl.ds(base, tok_per_w)], idx0_v)
    pltpu.sync_copy(pos_hbm.at[1, pl.ds(base, tok_per_w)], idx1_v)
    pltpu.sync_copy(x_hbm.at[pl.ds(base, tok_per_w)], rows_v)
    c0 = pltpu.async_copy(rows_v, xs_hbm.at[idx0_v], sem0)
    c1 = pltpu.async_copy(rows_v, xs_hbm.at[idx1_v], sem1)
    c0.wait()
    c1.wait()


def _gmm_body(gids_ref, nblk_ref, xs_ref, w1_ref, w3_ref, w2_ref, ys_ref,
              acc_ref, *, nf):
    f = pl.program_id(0)
    b = pl.program_id(1)
    bb = jnp.minimum(b, nblk_ref[0] - 1)

    @pl.when(b < nblk_ref[0])
    def _():
        xb = xs_ref[...].astype(jnp.float32)
        a = lax.dot_general(xb, w1_ref[0], (((1,), (1,)), ((), ())),
                            preferred_element_type=jnp.float32)
        b3 = lax.dot_general(xb, w3_ref[0], (((1,), (1,)), ((), ())),
                             preferred_element_type=jnp.float32)
        h = (a * jax.nn.sigmoid(a)) * b3
        y = lax.dot_general(h, w2_ref[0], (((1,), (1,)), ((), ())),
                            preferred_element_type=jnp.float32)
        blk = xs_ref.shape[0]
        if nf == 1:
            ys_ref[...] = y
        else:
            @pl.when(f == 0)
            def _():
                acc_ref[pl.ds(bb * blk, blk), :] = y

            @pl.when((f != 0) & (f != nf - 1))
            def _():
                acc_ref[pl.ds(bb * blk, blk), :] += y

            @pl.when(f == nf - 1)
            def _():
                ys_ref[...] = acc_ref[pl.ds(bb * blk, blk), :] + y


def _gather_body(ys_hbm, pos_hbm, y0_hbm, y1_hbm, rows_v, idx_v, sem,
                 *, tok_per_w, num_cores):
    wid = lax.axis_index("s") * num_cores + lax.axis_index("c")
    base = wid * tok_per_w
    pltpu.sync_copy(pos_hbm.at[0, pl.ds(base, tok_per_w)], idx_v)
    pltpu.async_copy(ys_hbm.at[idx_v], rows_v, sem).wait()
    pltpu.sync_copy(rows_v, y0_hbm.at[pl.ds(base, tok_per_w)])
    pltpu.sync_copy(pos_hbm.at[1, pl.ds(base, tok_per_w)], idx_v)
    pltpu.async_copy(ys_hbm.at[idx_v], rows_v, sem).wait()
    pltpu.sync_copy(rows_v, y1_hbm.at[pl.ds(base, tok_per_w)])


def _combine_body(y0_ref, y1_ref, pval_ref, out_ref):
    s = out_ref.shape[0]
    p0 = pval_ref[0, :].reshape(s, 1)
    p1 = pval_ref[1, :].reshape(s, 1)
    out_ref[...] = y0_ref[...] * p0 + y1_ref[...] * p1


def kernel(x, gate_w, w1, w2, w3):
    batch, seq, hidden = x.shape
    num_experts, ff, _ = w1.shape
    s = batch * seq
    x_flat = x.reshape(s, hidden)

    fblk = min(FBLK, ff)
    nf = ff // fblk
    nb_max = s * 2 // BLK + num_experts
    npad = nb_max * BLK

    pos, pval, gids, nblk, aux, xb16 = pl.pallas_call(
        functools.partial(_router_body, num_experts=num_experts,
                          nb_max=nb_max),
        in_specs=[
            pl.BlockSpec((s, hidden), lambda: (0, 0)),
            pl.BlockSpec((num_experts, hidden), lambda: (0, 0)),
        ],
        out_specs=[
            pl.BlockSpec((2, s), lambda: (0, 0)),
            pl.BlockSpec((2, s), lambda: (0, 0)),
            pl.BlockSpec((1, nb_max), lambda: (0, 0)),
            pl.BlockSpec((1, 1), lambda: (0, 0), memory_space=pltpu.SMEM),
            pl.BlockSpec((1, 1), lambda: (0, 0), memory_space=pltpu.SMEM),
            pl.BlockSpec((s, hidden), lambda: (0, 0)),
        ],
        out_shape=[
            jax.ShapeDtypeStruct((2, s), jnp.int32),
            jax.ShapeDtypeStruct((2, s), jnp.float32),
            jax.ShapeDtypeStruct((1, nb_max), jnp.int32),
            jax.ShapeDtypeStruct((1, 1), jnp.int32),
            jax.ShapeDtypeStruct((1, 1), jnp.float32),
            jax.ShapeDtypeStruct((s, hidden), jnp.bfloat16),
        ],
    )(x_flat, gate_w)

    hwords = hidden // 2
    x_words = jax.lax.bitcast_convert_type(
        xb16.reshape(s, hwords, 2), jnp.int32)
    info = plsc.get_sparse_core_info()
    nw = info.num_cores * info.num_subcores
    tok_per_w = s // nw
    mesh = plsc.VectorSubcoreMesh(core_axis_name="c", subcore_axis_name="s")
    xs_words = pl.kernel(
        functools.partial(_scatter_body, tok_per_w=tok_per_w,
                          num_cores=info.num_cores),
        out_type=jax.ShapeDtypeStruct((npad, hwords), jnp.int32),
        mesh=mesh,
        scratch_types=[
            pltpu.VMEM((tok_per_w, hwords), jnp.int32),
            pltpu.VMEM((tok_per_w,), jnp.int32),
            pltpu.VMEM((tok_per_w,), jnp.int32),
            pltpu.SemaphoreType.DMA,
            pltpu.SemaphoreType.DMA,
        ],
    )(x_words, pos)
    xs = jax.lax.bitcast_convert_type(
        xs_words, jnp.bfloat16).reshape(npad, hidden)

    def _xi(f, b, gids_ref, nblk_ref):
        return (jnp.minimum(b, nblk_ref[0] - 1), 0)

    def _w13i(f, b, gids_ref, nblk_ref):
        return (gids_ref[jnp.minimum(b, nblk_ref[0] - 1)], f, 0)

    def _w2i(f, b, gids_ref, nblk_ref):
        return (gids_ref[jnp.minimum(b, nblk_ref[0] - 1)], 0, f)

    def _oi(f, b, gids_ref, nblk_ref):
        return (jnp.where(f == nf - 1,
                          jnp.minimum(b, nblk_ref[0] - 1), 0), 0)

    ys = pl.pallas_call(
        functools.partial(_gmm_body, nf=nf),
        grid_spec=pltpu.PrefetchScalarGridSpec(
            num_scalar_prefetch=2,
            grid=(nf, nb_max),
            in_specs=[
                pl.BlockSpec((BLK, hidden), _xi),

                pl.BlockSpec((1, fblk, hidden), _w13i),
                pl.BlockSpec((1, fblk, hidden), _w13i),
                pl.BlockSpec((1, hidden, fblk), _w2i),
            ],
            out_specs=pl.BlockSpec((BLK, hidden), _oi),
            scratch_shapes=[pltpu.VMEM((npad, hidden), jnp.float32)],
        ),
        out_shape=jax.ShapeDtypeStruct((npad, hidden), jnp.float32),
        compiler_params=pltpu.CompilerParams(
            dimension_semantics=("arbitrary", "arbitrary")),
    )(gids.reshape(nb_max), nblk.reshape(1), xs, w1, w3, w2)

    y0, y1 = pl.kernel(
        functools.partial(_gather_body, tok_per_w=tok_per_w,
                          num_cores=info.num_cores),
        out_type=[
            jax.ShapeDtypeStruct((s, hidden), jnp.float32),
            jax.ShapeDtypeStruct((s, hidden), jnp.float32),
        ],
        mesh=mesh,
        scratch_types=[
            pltpu.VMEM((tok_per_w, hidden), jnp.float32),
            pltpu.VMEM((tok_per_w,), jnp.int32),
            pltpu.SemaphoreType.DMA,
        ],
    )(ys, pos)

    out = pl.pallas_call(
        _combine_body,
        in_specs=[
            pl.BlockSpec((s, hidden), lambda: (0, 0)),
            pl.BlockSpec((s, hidden), lambda: (0, 0)),
            pl.BlockSpec((2, s), lambda: (0, 0)),
        ],
        out_specs=pl.BlockSpec((s, hidden), lambda: (0, 0)),
        out_shape=jax.ShapeDtypeStruct((s, hidden), jnp.float32),
    )(y0, y1, pval)

    return out.reshape(batch, seq, hidden), aux.reshape(())


# speedup vs baseline: 1.5291x; 1.5291x over previous
import functools

import jax
import jax.numpy as jnp
from jax import lax
from jax.experimental import pallas as pl
from jax.experimental.pallas import tpu as pltpu
from jax.experimental.pallas import tpu_sc as plsc

BLK = 256
FBLK = 1024


def _router_body(x_ref, gw_ref, pos_ref, pval_ref, gids_ref, nblk_ref,
                 aux_ref, *, num_experts, nb_max):
    x = x_ref[...]
    s = x.shape[0]
    logits = lax.dot_general(x, gw_ref[...], (((1,), (1,)), ((), ())),
                             preferred_element_type=jnp.float32)
    iota_e = lax.broadcasted_iota(jnp.int32, (s, num_experts), 1)
    big = jnp.int32(num_experts + 1)
    m1 = jnp.max(logits, axis=1, keepdims=True)
    i1 = jnp.min(jnp.where(logits == m1, iota_e, big), axis=1, keepdims=True)
    masked = jnp.where(iota_e == i1, -jnp.inf, logits)
    m2 = jnp.max(masked, axis=1, keepdims=True)
    i2 = jnp.min(jnp.where(masked == m2, iota_e, big), axis=1, keepdims=True)
    p1 = jax.nn.sigmoid(m1 - m2)
    p2 = 1.0 - p1

    probs = jax.nn.softmax(logits, axis=1)
    colsum = jnp.sum(probs, axis=0)
    aux_ref[0, 0] = num_experts * jnp.sum(colsum * colsum) / (s * s)

    ind = (jnp.where(iota_e == i1, 1.0, 0.0)
           + jnp.where(iota_e == i2, 1.0, 0.0))
    ia = lax.broadcasted_iota(jnp.int32, (s, s), 0)
    ib = lax.broadcasted_iota(jnp.int32, (s, s), 1)
    ltri = jnp.where(ib < ia, 1.0, 0.0)
    rank = lax.dot_general(ltri, ind, (((1,), (0,)), ((), ())),
                           preferred_element_type=jnp.float32)
    cnt = jnp.sum(ind, axis=0, keepdims=True).astype(jnp.int32)
    blocks = (cnt + (BLK - 1)) // BLK
    ee = lax.broadcasted_iota(jnp.int32, (num_experts, num_experts), 0)
    ep = lax.broadcasted_iota(jnp.int32, (num_experts, num_experts), 1)
    excl = jnp.sum(jnp.where(ep < ee, blocks, 0), axis=1,
                   keepdims=True)
    off_rows = (excl.reshape(1, num_experts) * BLK).astype(jnp.float32)
    slot = off_rows + rank
    pos1 = jnp.sum(jnp.where(iota_e == i1, slot, 0.0), axis=1,
                   keepdims=True)
    pos2 = jnp.sum(jnp.where(iota_e == i2, slot, 0.0), axis=1,
                   keepdims=True)
    pos_ref[...] = jnp.concatenate(
        [pos1.reshape(1, s), pos2.reshape(1, s)], axis=0).astype(jnp.int32)
    pval_ref[...] = jnp.concatenate(
        [p1.reshape(1, s), p2.reshape(1, s)], axis=0)

    nblk_ref[0, 0] = jnp.sum(blocks)
    ends = (excl.reshape(num_experts, 1)
            + blocks.reshape(num_experts, 1))
    bb = lax.broadcasted_iota(jnp.int32, (num_experts, nb_max), 1)
    g = jnp.sum(jnp.where(bb >= ends, 1, 0), axis=0, keepdims=True)
    gids_ref[...] = jnp.minimum(g, num_experts - 1)


def _scatter_body(x_hbm, pos_hbm, xs_hbm, rows_v, idx0_v, idx1_v,
                  sem0, sem1, *, tok_per_w, num_cores):
    wid = lax.axis_index("s") * num_cores + lax.axis_index("c")
    base = wid * tok_per_w
    pltpu.sync_copy(pos_hbm.at[0, pl.ds(base, tok_per_w)], idx0_v)
    pltpu.sync_copy(pos_hbm.at[1, pl.ds(base, tok_per_w)], idx1_v)
    pltpu.sync_copy(x_hbm.at[pl.ds(base, tok_per_w)], rows_v)
    c0 = pltpu.async_copy(rows_v, xs_hbm.at[idx0_v], sem0)
    c1 = pltpu.async_copy(rows_v, xs_hbm.at[idx1_v], sem1)
    c0.wait()
    c1.wait()


def _gmm_body(gids_ref, nblk_ref, xs_ref, w1_ref, w3_ref, w2_ref, ys_ref,
              acc_ref, *, nf):
    f = pl.program_id(0)
    b = pl.program_id(1)
    bb = jnp.minimum(b, nblk_ref[0] - 1)

    @pl.when(b < nblk_ref[0])
    def _():
        xb = xs_ref[...]
        a = lax.dot_general(xb, w1_ref[0], (((1,), (1,)), ((), ())),
                            preferred_element_type=jnp.float32)
        b3 = lax.dot_general(xb, w3_ref[0], (((1,), (1,)), ((), ())),
                             preferred_element_type=jnp.float32)
        h = (a * jax.nn.sigmoid(a)) * b3
        y = lax.dot_general(h, w2_ref[0], (((1,), (1,)), ((), ())),
                            preferred_element_type=jnp.float32)
        blk = xs_ref.shape[0]
        if nf == 1:
            ys_ref[...] = y
        else:
            @pl.when(f == 0)
            def _():
                acc_ref[pl.ds(bb * blk, blk), :] = y

            @pl.when((f != 0) & (f != nf - 1))
            def _():
                acc_ref[pl.ds(bb * blk, blk), :] += y

            @pl.when(f == nf - 1)
            def _():
                ys_ref[...] = acc_ref[pl.ds(bb * blk, blk), :] + y


def _gather_body(ys_hbm, pos_hbm, y0_hbm, y1_hbm, rows_v, idx_v, sem,
                 *, tok_per_w, num_cores):
    wid = lax.axis_index("s") * num_cores + lax.axis_index("c")
    base = wid * tok_per_w
    pltpu.sync_copy(pos_hbm.at[0, pl.ds(base, tok_per_w)], idx_v)
    pltpu.async_copy(ys_hbm.at[idx_v], rows_v, sem).wait()
    pltpu.sync_copy(rows_v, y0_hbm.at[pl.ds(base, tok_per_w)])
    pltpu.sync_copy(pos_hbm.at[1, pl.ds(base, tok_per_w)], idx_v)
    pltpu.async_copy(ys_hbm.at[idx_v], rows_v, sem).wait()
    pltpu.sync_copy(rows_v, y1_hbm.at[pl.ds(base, tok_per_w)])


def _combine_body(y0_ref, y1_ref, pval_ref, out_ref):
    s = out_ref.shape[0]
    p0 = pval_ref[0, :].reshape(s, 1)
    p1 = pval_ref[1, :].reshape(s, 1)
    out_ref[...] = y0_ref[...] * p0 + y1_ref[...] * p1


def kernel(x, gate_w, w1, w2, w3):
    batch, seq, hidden = x.shape
    num_experts, ff, _ = w1.shape
    s = batch * seq
    x_flat = x.reshape(s, hidden)

    fblk = min(FBLK, ff)
    nf = ff // fblk
    nb_max = s * 2 // BLK + num_experts
    npad = nb_max * BLK

    pos, pval, gids, nblk, aux = pl.pallas_call(
        functools.partial(_router_body, num_experts=num_experts,
                          nb_max=nb_max),
        in_specs=[
            pl.BlockSpec((s, hidden), lambda: (0, 0)),
            pl.BlockSpec((num_experts, hidden), lambda: (0, 0)),
        ],
        out_specs=[
            pl.BlockSpec((2, s), lambda: (0, 0)),
            pl.BlockSpec((2, s), lambda: (0, 0)),
            pl.BlockSpec((1, nb_max), lambda: (0, 0)),
            pl.BlockSpec((1, 1), lambda: (0, 0), memory_space=pltpu.SMEM),
            pl.BlockSpec((1, 1), lambda: (0, 0), memory_space=pltpu.SMEM),
        ],
        out_shape=[
            jax.ShapeDtypeStruct((2, s), jnp.int32),
            jax.ShapeDtypeStruct((2, s), jnp.float32),
            jax.ShapeDtypeStruct((1, nb_max), jnp.int32),
            jax.ShapeDtypeStruct((1, 1), jnp.int32),
            jax.ShapeDtypeStruct((1, 1), jnp.float32),
        ],
    )(x_flat, gate_w)

    info = plsc.get_sparse_core_info()
    nw = info.num_cores * info.num_subcores
    tok_per_w = s // nw
    mesh = plsc.VectorSubcoreMesh(core_axis_name="c", subcore_axis_name="s")
    xs = pl.kernel(
        functools.partial(_scatter_body, tok_per_w=tok_per_w,
                          num_cores=info.num_cores),
        out_type=jax.ShapeDtypeStruct((npad, hidden), jnp.float32),
        mesh=mesh,
        scratch_types=[
            pltpu.VMEM((tok_per_w, hidden), jnp.float32),
            pltpu.VMEM((tok_per_w,), jnp.int32),
            pltpu.VMEM((tok_per_w,), jnp.int32),
            pltpu.SemaphoreType.DMA,
            pltpu.SemaphoreType.DMA,
        ],
    )(x_flat, pos)

    def _xi(f, b, gids_ref, nblk_ref):
        return (jnp.minimum(b, nblk_ref[0] - 1), 0)

    def _w13i(f, b, gids_ref, nblk_ref):
        return (gids_ref[jnp.minimum(b, nblk_ref[0] - 1)], f, 0)

    def _w2i(f, b, gids_ref, nblk_ref):
        return (gids_ref[jnp.minimum(b, nblk_ref[0] - 1)], 0, f)

    def _oi(f, b, gids_ref, nblk_ref):
        return (jnp.where(f == nf - 1,
                          jnp.minimum(b, nblk_ref[0] - 1), 0), 0)

    ys = pl.pallas_call(
        functools.partial(_gmm_body, nf=nf),
        grid_spec=pltpu.PrefetchScalarGridSpec(
            num_scalar_prefetch=2,
            grid=(nf, nb_max),
            in_specs=[
                pl.BlockSpec((BLK, hidden), _xi),
                pl.BlockSpec((1, fblk, hidden), _w13i),
                pl.BlockSpec((1, fblk, hidden), _w13i),
                pl.BlockSpec((1, hidden, fblk), _w2i),
            ],
            out_specs=pl.BlockSpec((BLK, hidden), _oi),
            scratch_shapes=[pltpu.VMEM((npad, hidden), jnp.float32)],
        ),
        out_shape=jax.ShapeDtypeStruct((npad, hidden), jnp.float32),
        compiler_params=pltpu.CompilerParams(
            dimension_semantics=("arbitrary", "arbitrary")),
    )(gids.reshape(nb_max), nblk.reshape(1), xs, w1, w3, w2)

    y0, y1 = pl.kernel(
        functools.partial(_gather_body, tok_per_w=tok_per_w,
                          num_cores=info.num_cores),
        out_type=[
            jax.ShapeDtypeStruct((s, hidden), jnp.float32),
            jax.ShapeDtypeStruct((s, hidden), jnp.float32),
        ],
        mesh=mesh,
        scratch_types=[
            pltpu.VMEM((tok_per_w, hidden), jnp.float32),
            pltpu.VMEM((tok_per_w,), jnp.int32),
            pltpu.SemaphoreType.DMA,
        ],
    )(ys, pos)

    out = pl.pallas_call(
        _combine_body,
        in_specs=[
            pl.BlockSpec((s, hidden), lambda: (0, 0)),
            pl.BlockSpec((s, hidden), lambda: (0, 0)),
            pl.BlockSpec((2, s), lambda: (0, 0)),
        ],
        out_specs=pl.BlockSpec((s, hidden), lambda: (0, 0)),
        out_shape=jax.ShapeDtypeStruct((s, hidden), jnp.float32),
    )(y0, y1, pval)

    return out.reshape(batch, seq, hidden), aux.reshape(())


# speedup vs baseline: 1.5584x; 1.0192x over previous
import functools

import jax
import jax.numpy as jnp
from jax import lax
from jax.experimental import pallas as pl
from jax.experimental.pallas import tpu as pltpu
from jax.experimental.pallas import tpu_sc as plsc

BLK = 256
FBLK = 1024


def _router_body(x_ref, gw_ref, pos_ref, pval_ref, gids_ref, nblk_ref,
                 aux_ref, *, num_experts, nb_max):
    x = x_ref[...]
    s = x.shape[0]
    logits = lax.dot_general(x, gw_ref[...], (((1,), (1,)), ((), ())),
                             preferred_element_type=jnp.float32)
    iota_e = lax.broadcasted_iota(jnp.int32, (s, num_experts), 1)
    big = jnp.int32(num_experts + 1)
    m1 = jnp.max(logits, axis=1, keepdims=True)
    i1 = jnp.min(jnp.where(logits == m1, iota_e, big), axis=1, keepdims=True)
    masked = jnp.where(iota_e == i1, -jnp.inf, logits)
    m2 = jnp.max(masked, axis=1, keepdims=True)
    i2 = jnp.min(jnp.where(masked == m2, iota_e, big), axis=1, keepdims=True)
    p1 = jax.nn.sigmoid(m1 - m2)
    p2 = 1.0 - p1

    probs = jax.nn.softmax(logits, axis=1)
    colsum = jnp.sum(probs, axis=0)
    aux_ref[0, 0] = num_experts * jnp.sum(colsum * colsum) / (s * s)

    ind = (jnp.where(iota_e == i1, 1.0, 0.0)
           + jnp.where(iota_e == i2, 1.0, 0.0))
    ia = lax.broadcasted_iota(jnp.int32, (s, s), 0)
    ib = lax.broadcasted_iota(jnp.int32, (s, s), 1)
    ltri = jnp.where(ib < ia, 1.0, 0.0)
    rank = lax.dot_general(ltri, ind, (((1,), (0,)), ((), ())),
                           preferred_element_type=jnp.float32)
    cnt = jnp.sum(ind, axis=0, keepdims=True).astype(jnp.int32)
    blocks = (cnt + (BLK - 1)) // BLK
    ee = lax.broadcasted_iota(jnp.int32, (num_experts, num_experts), 0)
    ep = lax.broadcasted_iota(jnp.int32, (num_experts, num_experts), 1)
    excl = jnp.sum(jnp.where(ep < ee, blocks, 0), axis=1,
                   keepdims=True)
    off_rows = (excl.reshape(1, num_experts) * BLK).astype(jnp.float32)
    slot = off_rows + rank
    pos1 = jnp.sum(jnp.where(iota_e == i1, slot, 0.0), axis=1,
                   keepdims=True)
    pos2 = jnp.sum(jnp.where(iota_e == i2, slot, 0.0), axis=1,
                   keepdims=True)
    pos_ref[...] = jnp.concatenate(
        [pos1.reshape(1, s), pos2.reshape(1, s)], axis=0).astype(jnp.int32)
    pval_ref[...] = jnp.concatenate(
        [p1.reshape(1, s), p2.reshape(1, s)], axis=0)

    nblk_ref[0, 0] = jnp.sum(blocks)
    ends = (excl.reshape(num_experts, 1)
            + blocks.reshape(num_experts, 1))
    bb = lax.broadcasted_iota(jnp.int32, (num_experts, nb_max), 1)
    g = jnp.sum(jnp.where(bb >= ends, 1, 0), axis=0, keepdims=True)
    gids_ref[...] = jnp.minimum(g, num_experts - 1)


def _scatter_body(x_hbm, pos_hbm, xs_hbm, rows_v, idx0_v, idx1_v,
                  sem0, sem1, *, tok_per_w, num_cores):
    wid = lax.axis_index("s") * num_cores + lax.axis_index("c")
    base = wid * tok_per_w
    pltpu.sync_copy(pos_hbm.at[0, pl.ds(base, tok_per_w)], idx0_v)
    pltpu.sync_copy(pos_hbm.at[1, pl.ds(base, tok_per_w)], idx1_v)
    pltpu.sync_copy(x_hbm.at[pl.ds(base, tok_per_w)], rows_v)
    c0 = pltpu.async_copy(rows_v, xs_hbm.at[idx0_v], sem0)
    c1 = pltpu.async_copy(rows_v, xs_hbm.at[idx1_v], sem1)
    c0.wait()
    c1.wait()


def _gmm_body(gids_ref, nblk_ref, xs_ref, w1_ref, w3_ref, w2_ref, ys_ref,
              acc_ref, *, nf):
    f = pl.program_id(0)
    b = pl.program_id(1)
    bb = jnp.minimum(b, nblk_ref[0] - 1)

    @pl.when(b < nblk_ref[0])
    def _():
        blk_ = ys_ref.shape[0]
        xb = xs_ref[pl.ds(bb * blk_, blk_), :]
        a = lax.dot_general(xb, w1_ref[0], (((1,), (1,)), ((), ())),
                            preferred_element_type=jnp.float32)
        b3 = lax.dot_general(xb, w3_ref[0], (((1,), (1,)), ((), ())),
                             preferred_element_type=jnp.float32)
        h = (a * jax.nn.sigmoid(a)) * b3
        y = lax.dot_general(h, w2_ref[0], (((1,), (1,)), ((), ())),
                            preferred_element_type=jnp.float32)
        blk = ys_ref.shape[0]
        if nf == 1:
            ys_ref[...] = y
        else:
            @pl.when(f == 0)
            def _():
                acc_ref[pl.ds(bb * blk, blk), :] = y.astype(acc_ref.dtype)

            @pl.when((f != 0) & (f != nf - 1))
            def _():
                acc_ref[pl.ds(bb * blk, blk), :] = (
                    acc_ref[pl.ds(bb * blk, blk), :].astype(jnp.float32) + y
                ).astype(acc_ref.dtype)

            @pl.when(f == nf - 1)
            def _():
                ys_ref[...] = (
                    acc_ref[pl.ds(bb * blk, blk), :].astype(jnp.float32) + y)


def _gather_body(ys_hbm, pos_hbm, y0_hbm, y1_hbm, rows_v, idx_v, sem,
                 *, tok_per_w, num_cores):
    wid = lax.axis_index("s") * num_cores + lax.axis_index("c")
    base = wid * tok_per_w
    pltpu.sync_copy(pos_hbm.at[0, pl.ds(base, tok_per_w)], idx_v)
    pltpu.async_copy(ys_hbm.at[idx_v], rows_v, sem).wait()
    pltpu.sync_copy(rows_v, y0_hbm.at[pl.ds(base, tok_per_w)])
    pltpu.sync_copy(pos_hbm.at[1, pl.ds(base, tok_per_w)], idx_v)
    pltpu.async_copy(ys_hbm.at[idx_v], rows_v, sem).wait()
    pltpu.sync_copy(rows_v, y1_hbm.at[pl.ds(base, tok_per_w)])


def _combine_body(y0_ref, y1_ref, pval_ref, out_ref):
    s = out_ref.shape[0]
    p0 = pval_ref[0, :].reshape(s, 1)
    p1 = pval_ref[1, :].reshape(s, 1)
    out_ref[...] = y0_ref[...] * p0 + y1_ref[...] * p1


def kernel(x, gate_w, w1, w2, w3):
    batch, seq, hidden = x.shape
    num_experts, ff, _ = w1.shape
    s = batch * seq
    x_flat = x.reshape(s, hidden)

    fblk = min(FBLK, ff)
    nf = ff // fblk
    nb_max = s * 2 // BLK + num_experts
    npad = nb_max * BLK

    pos, pval, gids, nblk, aux = pl.pallas_call(
        functools.partial(_router_body, num_experts=num_experts,
                          nb_max=nb_max),
        in_specs=[
            pl.BlockSpec((s, hidden), lambda: (0, 0)),
            pl.BlockSpec((num_experts, hidden), lambda: (0, 0)),
        ],
        out_specs=[
            pl.BlockSpec((2, s), lambda: (0, 0)),
            pl.BlockSpec((2, s), lambda: (0, 0)),
            pl.BlockSpec((1, nb_max), lambda: (0, 0)),
            pl.BlockSpec((1, 1), lambda: (0, 0), memory_space=pltpu.SMEM),
            pl.BlockSpec((1, 1), lambda: (0, 0), memory_space=pltpu.SMEM),
        ],
        out_shape=[
            jax.ShapeDtypeStruct((2, s), jnp.int32),
            jax.ShapeDtypeStruct((2, s), jnp.float32),
            jax.ShapeDtypeStruct((1, nb_max), jnp.int32),
            jax.ShapeDtypeStruct((1, 1), jnp.int32),
            jax.ShapeDtypeStruct((1, 1), jnp.float32),
        ],
    )(x_flat, gate_w)

    info = plsc.get_sparse_core_info()
    nw = info.num_cores * info.num_subcores
    tok_per_w = s // nw
    mesh = plsc.VectorSubcoreMesh(core_axis_name="c", subcore_axis_name="s")
    xs = pl.kernel(
        functools.partial(_scatter_body, tok_per_w=tok_per_w,
                          num_cores=info.num_cores),
        out_type=jax.ShapeDtypeStruct((npad, hidden), jnp.float32),
        mesh=mesh,
        scratch_types=[
            pltpu.VMEM((tok_per_w, hidden), jnp.float32),
            pltpu.VMEM((tok_per_w,), jnp.int32),
            pltpu.VMEM((tok_per_w,), jnp.int32),
            pltpu.SemaphoreType.DMA,
            pltpu.SemaphoreType.DMA,
        ],
    )(x_flat, pos)

    def _xi(f, b, gids_ref, nblk_ref):
        return (jnp.minimum(b, nblk_ref[0] - 1), 0)

    def _w13i(f, b, gids_ref, nblk_ref):
        return (gids_ref[jnp.minimum(b, nblk_ref[0] - 1)], f, 0)

    def _w2i(f, b, gids_ref, nblk_ref):
        return (gids_ref[jnp.minimum(b, nblk_ref[0] - 1)], 0, f)

    def _oi(f, b, gids_ref, nblk_ref):
        return (jnp.where(f == nf - 1,
                          jnp.minimum(b, nblk_ref[0] - 1), 0), 0)

    ys = pl.pallas_call(
        functools.partial(_gmm_body, nf=nf),
        grid_spec=pltpu.PrefetchScalarGridSpec(
            num_scalar_prefetch=2,
            grid=(nf, nb_max),
            in_specs=[
                pl.BlockSpec((npad, hidden), lambda f, b, g, n: (0, 0),
                             pipeline_mode=pl.Buffered(buffer_count=1)),
                pl.BlockSpec((1, fblk, hidden), _w13i),
                pl.BlockSpec((1, fblk, hidden), _w13i),
                pl.BlockSpec((1, hidden, fblk), _w2i),
            ],
            out_specs=pl.BlockSpec((BLK, hidden), _oi),
            scratch_shapes=[pltpu.VMEM((npad, hidden), jnp.bfloat16)],
        ),
        out_shape=jax.ShapeDtypeStruct((npad, hidden), jnp.float32),
        compiler_params=pltpu.CompilerParams(
            dimension_semantics=("arbitrary", "arbitrary"),
            vmem_limit_bytes=110 * 1024 * 1024),
    )(gids.reshape(nb_max), nblk.reshape(1), xs, w1, w3, w2)

    y0, y1 = pl.kernel(
        functools.partial(_gather_body, tok_per_w=tok_per_w,
                          num_cores=info.num_cores),
        out_type=[
            jax.ShapeDtypeStruct((s, hidden), jnp.float32),
            jax.ShapeDtypeStruct((s, hidden), jnp.float32),
        ],
        mesh=mesh,
        scratch_types=[
            pltpu.VMEM((tok_per_w, hidden), jnp.float32),
            pltpu.VMEM((tok_per_w,), jnp.int32),
            pltpu.SemaphoreType.DMA,
        ],
    )(ys, pos)

    out = pl.pallas_call(
        _combine_body,
        in_specs=[
            pl.BlockSpec((s, hidden), lambda: (0, 0)),
            pl.BlockSpec((s, hidden), lambda: (0, 0)),
            pl.BlockSpec((2, s), lambda: (0, 0)),
        ],
        out_specs=pl.BlockSpec((s, hidden), lambda: (0, 0)),
        out_shape=jax.ShapeDtypeStruct((s, hidden), jnp.float32),
    )(y0, y1, pval)

    return out.reshape(batch, seq, hidden), aux.reshape(())


# speedup vs baseline: 1.5585x; 1.0001x over previous
import functools

import jax
import jax.numpy as jnp
from jax import lax
from jax.experimental import pallas as pl
from jax.experimental.pallas import tpu as pltpu
from jax.experimental.pallas import tpu_sc as plsc

BLK = 256
FBLK = 1024


def _router_body(x_ref, gw_ref, pos_ref, pval_ref, gids_ref, nblk_ref,
                 aux_ref, *, num_experts, nb_max):
    x = x_ref[...]
    s = x.shape[0]
    logits = lax.dot_general(x, gw_ref[...], (((1,), (1,)), ((), ())),
                             preferred_element_type=jnp.float32)
    iota_e = lax.broadcasted_iota(jnp.int32, (s, num_experts), 1)
    big = jnp.int32(num_experts + 1)
    m1 = jnp.max(logits, axis=1, keepdims=True)
    i1 = jnp.min(jnp.where(logits == m1, iota_e, big), axis=1, keepdims=True)
    masked = jnp.where(iota_e == i1, -jnp.inf, logits)
    m2 = jnp.max(masked, axis=1, keepdims=True)
    i2 = jnp.min(jnp.where(masked == m2, iota_e, big), axis=1, keepdims=True)
    p1 = jax.nn.sigmoid(m1 - m2)
    p2 = 1.0 - p1

    probs = jax.nn.softmax(logits, axis=1)
    colsum = jnp.sum(probs, axis=0)
    aux_ref[0, 0] = num_experts * jnp.sum(colsum * colsum) / (s * s)

    ind = (jnp.where(iota_e == i1, 1.0, 0.0)
           + jnp.where(iota_e == i2, 1.0, 0.0))
    ia = lax.broadcasted_iota(jnp.int32, (s, s), 0)
    ib = lax.broadcasted_iota(jnp.int32, (s, s), 1)
    ltri = jnp.where(ib < ia, 1.0, 0.0)
    rank = lax.dot_general(ltri, ind, (((1,), (0,)), ((), ())),
                           preferred_element_type=jnp.float32)
    cnt = jnp.sum(ind, axis=0, keepdims=True).astype(jnp.int32)
    blocks = (cnt + (BLK - 1)) // BLK
    ee = lax.broadcasted_iota(jnp.int32, (num_experts, num_experts), 0)
    ep = lax.broadcasted_iota(jnp.int32, (num_experts, num_experts), 1)
    excl = jnp.sum(jnp.where(ep < ee, blocks, 0), axis=1,
                   keepdims=True)
    off_rows = (excl.reshape(1, num_experts) * BLK).astype(jnp.float32)
    slot = off_rows + rank
    pos1 = jnp.sum(jnp.where(iota_e == i1, slot, 0.0), axis=1,
                   keepdims=True)
    pos2 = jnp.sum(jnp.where(iota_e == i2, slot, 0.0), axis=1,
                   keepdims=True)
    pos_ref[...] = jnp.concatenate(
        [pos1.reshape(1, s), pos2.reshape(1, s)], axis=0).astype(jnp.int32)
    pval_ref[...] = jnp.concatenate(
        [p1.reshape(1, s), p2.reshape(1, s)], axis=0)

    nblk_ref[0, 0] = jnp.sum(blocks)
    ends = (excl.reshape(num_experts, 1)
            + blocks.reshape(num_experts, 1))
    bb = lax.broadcasted_iota(jnp.int32, (num_experts, nb_max), 1)
    g = jnp.sum(jnp.where(bb >= ends, 1, 0), axis=0, keepdims=True)
    gids_ref[...] = jnp.minimum(g, num_experts - 1)


def _scatter_body(x_hbm, pos_hbm, xs_hbm, rows_v, idx0_v, idx1_v,
                  sem0, sem1, *, tok_per_w, num_cores):
    wid = lax.axis_index("s") * num_cores + lax.axis_index("c")
    base = wid * tok_per_w
    pltpu.sync_copy(pos_hbm.at[0, pl.ds(base, tok_per_w)], idx0_v)
    pltpu.sync_copy(pos_hbm.at[1, pl.ds(base, tok_per_w)], idx1_v)
    pltpu.sync_copy(x_hbm.at[pl.ds(base, tok_per_w)], rows_v)
    c0 = pltpu.async_copy(rows_v, xs_hbm.at[idx0_v], sem0)
    c1 = pltpu.async_copy(rows_v, xs_hbm.at[idx1_v], sem1)
    c0.wait()
    c1.wait()


def _gmm_body(gids_ref, nblk_ref, xs_ref, w1_ref, w3_ref, w2_ref, ys_ref,
              acc_ref, *, nf):
    f = pl.program_id(0)
    b = pl.program_id(1)
    bb = jnp.minimum(b, nblk_ref[0] - 1)

    @pl.when(b < nblk_ref[0])
    def _():
        blk = ys_ref.shape[0]
        xb = xs_ref[pl.ds(bb * blk, blk), :]
        a = lax.dot_general(xb, w1_ref[0], (((1,), (1,)), ((), ())),
                            preferred_element_type=jnp.float32)
        b3 = lax.dot_general(xb, w3_ref[0], (((1,), (1,)), ((), ())),
                             preferred_element_type=jnp.float32)
        h = (a * jax.nn.sigmoid(a)) * b3
        y = lax.dot_general(h, w2_ref[0], (((1,), (1,)), ((), ())),
                            preferred_element_type=jnp.float32)
        if nf == 1:
            ys_ref[...] = y
        else:
            @pl.when(f == 0)
            def _():
                acc_ref[pl.ds(bb * blk, blk), :] = y.astype(acc_ref.dtype)

            @pl.when((f != 0) & (f != nf - 1))
            def _():
                acc_ref[pl.ds(bb * blk, blk), :] = (
                    acc_ref[pl.ds(bb * blk, blk), :].astype(jnp.float32) + y
                ).astype(acc_ref.dtype)

            @pl.when(f == nf - 1)
            def _():
                ys_ref[...] = (
                    acc_ref[pl.ds(bb * blk, blk), :].astype(jnp.float32) + y)


def _gather_body(ys_hbm, pos_hbm, y0_hbm, y1_hbm, rows_v, idx_v, sem,
                 *, tok_per_w, num_cores):
    wid = lax.axis_index("s") * num_cores + lax.axis_index("c")
    base = wid * tok_per_w
    pltpu.sync_copy(pos_hbm.at[0, pl.ds(base, tok_per_w)], idx_v)
    pltpu.async_copy(ys_hbm.at[idx_v], rows_v, sem).wait()
    pltpu.sync_copy(rows_v, y0_hbm.at[pl.ds(base, tok_per_w)])
    pltpu.sync_copy(pos_hbm.at[1, pl.ds(base, tok_per_w)], idx_v)
    pltpu.async_copy(ys_hbm.at[idx_v], rows_v, sem).wait()
    pltpu.sync_copy(rows_v, y1_hbm.at[pl.ds(base, tok_per_w)])


def _combine_body(y0_ref, y1_ref, pval_ref, out_ref):
    s = out_ref.shape[0]
    p0 = pval_ref[0, :].reshape(s, 1)
    p1 = pval_ref[1, :].reshape(s, 1)
    out_ref[...] = y0_ref[...] * p0 + y1_ref[...] * p1


def kernel(x, gate_w, w1, w2, w3):
    batch, seq, hidden = x.shape
    num_experts, ff, _ = w1.shape
    s = batch * seq
    x_flat = x.reshape(s, hidden)

    fblk = min(FBLK, ff)
    nf = ff // fblk
    nb_max = s * 2 // BLK + num_experts
    npad = nb_max * BLK

    pos, pval, gids, nblk, aux = pl.pallas_call(
        functools.partial(_router_body, num_experts=num_experts,
                          nb_max=nb_max),
        in_specs=[
            pl.BlockSpec((s, hidden), lambda: (0, 0)),
            pl.BlockSpec((num_experts, hidden), lambda: (0, 0)),
        ],
        out_specs=[
            pl.BlockSpec((2, s), lambda: (0, 0)),
            pl.BlockSpec((2, s), lambda: (0, 0)),
            pl.BlockSpec((1, nb_max), lambda: (0, 0)),
            pl.BlockSpec((1, 1), lambda: (0, 0), memory_space=pltpu.SMEM),
            pl.BlockSpec((1, 1), lambda: (0, 0), memory_space=pltpu.SMEM),
        ],
        out_shape=[
            jax.ShapeDtypeStruct((2, s), jnp.int32),
            jax.ShapeDtypeStruct((2, s), jnp.float32),
            jax.ShapeDtypeStruct((1, nb_max), jnp.int32),
            jax.ShapeDtypeStruct((1, 1), jnp.int32),
            jax.ShapeDtypeStruct((1, 1), jnp.float32),
        ],
    )(x_flat, gate_w)

    info = plsc.get_sparse_core_info()
    nw = info.num_cores * info.num_subcores
    tok_per_w = s // nw
    mesh = plsc.VectorSubcoreMesh(core_axis_name="c", subcore_axis_name="s")
    xs = pl.kernel(
        functools.partial(_scatter_body, tok_per_w=tok_per_w,
                          num_cores=info.num_cores),
        out_type=jax.ShapeDtypeStruct((npad, hidden), jnp.float32),
        mesh=mesh,
        scratch_types=[
            pltpu.VMEM((tok_per_w, hidden), jnp.float32),
            pltpu.VMEM((tok_per_w,), jnp.int32),
            pltpu.VMEM((tok_per_w,), jnp.int32),
            pltpu.SemaphoreType.DMA,
            pltpu.SemaphoreType.DMA,
        ],
    )(x_flat, pos)

    def _w13i(f, b, gids_ref, nblk_ref):
        return (gids_ref[jnp.minimum(b, nblk_ref[0] - 1)], f, 0)

    def _w2i(f, b, gids_ref, nblk_ref):
        return (gids_ref[jnp.minimum(b, nblk_ref[0] - 1)], 0, f)

    def _oi(f, b, gids_ref, nblk_ref):
        return (jnp.where(f == nf - 1,
                          jnp.minimum(b, nblk_ref[0] - 1), 0), 0)

    ys = pl.pallas_call(
        functools.partial(_gmm_body, nf=nf),
        grid_spec=pltpu.PrefetchScalarGridSpec(
            num_scalar_prefetch=2,
            grid=(nf, nb_max),
            in_specs=[
                pl.BlockSpec((npad, hidden), lambda f, b, g, n: (0, 0),
                             pipeline_mode=pl.Buffered(buffer_count=1)),
                pl.BlockSpec((1, fblk, hidden), _w13i),
                pl.BlockSpec((1, fblk, hidden), _w13i),
                pl.BlockSpec((1, hidden, fblk), _w2i),
            ],
            out_specs=pl.BlockSpec((BLK, hidden), _oi),
            scratch_shapes=[pltpu.VMEM((npad, hidden), jnp.bfloat16)],
        ),
        out_shape=jax.ShapeDtypeStruct((npad, hidden), jnp.float32),
        compiler_params=pltpu.CompilerParams(
            dimension_semantics=("arbitrary", "arbitrary"),
            vmem_limit_bytes=110 * 1024 * 1024),
    )(gids.reshape(nb_max), nblk.reshape(1), xs, w1, w3, w2)

    y0, y1 = pl.kernel(
        functools.partial(_gather_body, tok_per_w=tok_per_w,
                          num_cores=info.num_cores),
        out_type=[
            jax.ShapeDtypeStruct((s, hidden), jnp.float32),
            jax.ShapeDtypeStruct((s, hidden), jnp.float32),
        ],
        mesh=mesh,
        scratch_types=[
            pltpu.VMEM((tok_per_w, hidden), jnp.float32),
            pltpu.VMEM((tok_per_w,), jnp.int32),
            pltpu.SemaphoreType.DMA,
        ],
    )(ys, pos)

    out = pl.pallas_call(
        _combine_body,
        in_specs=[
            pl.BlockSpec((s, hidden), lambda: (0, 0)),
            pl.BlockSpec((s, hidden), lambda: (0, 0)),
            pl.BlockSpec((2, s), lambda: (0, 0)),
        ],
        out_specs=pl.BlockSpec((s, hidden), lambda: (0, 0)),
        out_shape=jax.ShapeDtypeStruct((s, hidden), jnp.float32),
    )(y0, y1, pval)

    return out.reshape(batch, seq, hidden), aux.reshape(())
